# Initial kernel scaffold; baseline (speedup 1.0000x reference)
import jax
import jax.numpy as jnp
from jax.experimental import pallas as pl


def kernel(x, edge_index, Wl1, bl1, Wr1, gamma, beta, Wl2, bl2, Wr2):
    raise NotImplementedError("write your pallas kernel here")


# speedup vs baseline: 5.4335x; 5.4335x over previous
import functools

import jax
import jax.numpy as jnp
from jax import lax
from jax.experimental import pallas as pl
from jax.experimental.pallas import tpu as pltpu
from jax.experimental.pallas import tpu_sc as plsc

N = 10000
E = 320000
D = 128
HD = D // 2
NC = 2
NS = 16
EPT = E // NS
K = 80
NB = EPT // K
NPAD = 10240
RPT = NPAD // NS
ZCH = RPT // 5
CW = 16


def _sc_aggregate(x2, src3, src3p, dst3, with_counts):
    mesh = plsc.VectorSubcoreMesh(core_axis_name="c", subcore_axis_name="s")

    @functools.partial(
        pl.kernel,
        out_type=[
            jax.ShapeDtypeStruct((NC, NPAD, HD), jnp.float32),
            jax.ShapeDtypeStruct((NPAD, CW), jnp.float32),
        ],
        mesh=mesh,
        compiler_params=pltpu.CompilerParams(use_tc_tiling_on_sc=False),
        scratch_types=[
            pltpu.VMEM((NB, K), jnp.int32),
            pltpu.VMEM((NB, K), jnp.int32),
            pltpu.VMEM((K, HD), jnp.float32),
            pltpu.VMEM((K, CW), jnp.float32),
            pltpu.VMEM((ZCH, HD), jnp.float32),
            pltpu.VMEM((RPT, CW), jnp.float32),
            pltpu.VMEM_SHARED((NPAD, HD), jnp.float32),
            pltpu.VMEM_SHARED((NPAD, CW), jnp.float32),
        ],
    )
    def agg_kernel(x_hbm, src_hbm, srcp_hbm, dst_hbm, out_hbm, outc_hbm,
                   srcv, dstv, rows, ones, zrow, zcnt, acc_s, cnt_s):
        c = lax.axis_index("c")
        s = lax.axis_index("s")

        def fill_zrow(i, _):
            for j in range(HD // 16):
                zrow[i, pl.ds(j * 16, 16)] = jnp.zeros((16,), jnp.float32)
            return 0
        lax.fori_loop(0, ZCH, fill_zrow, 0)

        if with_counts:
            def fill_zcnt(i, _):
                zcnt[i, :] = jnp.zeros((16,), jnp.float32)
                return 0
            lax.fori_loop(0, RPT, fill_zcnt, 0)

            def fill_ones(i, _):
                ones[i, :] = jnp.ones((16,), jnp.float32)
                return 0
            lax.fori_loop(0, K, fill_ones, 0)

        base = s * RPT
        for z in range(5):
            pltpu.sync_copy(zrow, acc_s.at[pl.ds(base + z * ZCH, ZCH)])
        if with_counts:
            @pl.when(c == 0)
            def _():
                pltpu.sync_copy(zcnt, cnt_s.at[pl.ds(base, RPT)])

        @pl.when(c == 0)
        def _():
            pltpu.sync_copy(src_hbm.at[s], srcv)

        @pl.when(c == 1)
        def _():
            pltpu.sync_copy(srcp_hbm.at[s], srcv)

        pltpu.sync_copy(dst_hbm.at[s], dstv)

        plsc.subcore_barrier()

        if with_counts:
            def body(i, _):
                pltpu.sync_copy(x_hbm.at[srcv.at[i]], rows)
                pltpu.sync_copy(rows, acc_s.at[dstv.at[i]], add=True)

                @pl.when(c == 0)
                def _():
                    pltpu.sync_copy(ones, cnt_s.at[dstv.at[i]], add=True)
                return 0
        else:
            def body(i, _):
                pltpu.sync_copy(x_hbm.at[srcv.at[i]], rows)
                pltpu.sync_copy(rows, acc_s.at[dstv.at[i]], add=True)
                return 0
        lax.fori_loop(0, NB, body, 0)

        plsc.subcore_barrier()
        pltpu.sync_copy(acc_s.at[pl.ds(base, RPT)],
                        out_hbm.at[c, pl.ds(base, RPT)])
        if with_counts:
            @pl.when(c == 0)
            def _():
                pltpu.sync_copy(cnt_s.at[pl.ds(base, RPT)],
                                outc_hbm.at[pl.ds(base, RPT)])

    return agg_kernel(x2, src3, src3p, dst3)


def _split_stack(h):
    return jnp.concatenate([h[:, :HD], h[:, HD:]], axis=0)


def _tc_layer1(S, C, x, Wl1, bl1, Wr1, gamma, beta, Wr2, bl2):
    def body(S_ref, C_ref, x_ref, Wl1_ref, bl1_ref, Wr1_ref, g_ref, b_ref,
             Wr2_ref, bl2_ref, h2_ref, r2_ref):
        inv = 1.0 / jnp.maximum(C_ref[:N, 0:1], 1.0)
        aggL = S_ref[0, :N, :] * inv
        aggR = S_ref[1, :N, :] * inv
        h = (jnp.dot(aggL, Wl1_ref[:HD, :],
                     preferred_element_type=jnp.float32)
             + jnp.dot(aggR, Wl1_ref[HD:, :],
                       preferred_element_type=jnp.float32)
             + jnp.dot(x_ref[...], Wr1_ref[...],
                       preferred_element_type=jnp.float32)
             + bl1_ref[...])
        mu = jnp.mean(h, axis=0, keepdims=True)
        var = jnp.mean((h - mu) * (h - mu), axis=0, keepdims=True)
        hn = (h - mu) / jnp.sqrt(var + 1e-5) * g_ref[...] + b_ref[...]
        h2 = jnp.where(hn >= 0, hn, 0.01 * hn)
        h2_ref[...] = h2
        r2_ref[...] = (jnp.dot(h2, Wr2_ref[...],
                               preferred_element_type=jnp.float32)
                       + bl2_ref[...])

    return pl.pallas_call(
        body,
        out_shape=[
            jax.ShapeDtypeStruct((N, D), jnp.float32),
            jax.ShapeDtypeStruct((N, D), jnp.float32),
        ],
    )(S, C, x, Wl1, bl1, Wr1, gamma, beta, Wr2, bl2)


def _tc_layer2(S2, C, r2, Wl2):
    def body(S_ref, C_ref, r2_ref, Wl2_ref, out_ref):
        inv = 1.0 / jnp.maximum(C_ref[:N, 0:1], 1.0)
        aggL = S_ref[0, :N, :] * inv
        aggR = S_ref[1, :N, :] * inv
        out_ref[...] = (jnp.dot(aggL, Wl2_ref[:HD, :],
                                preferred_element_type=jnp.float32)
                        + jnp.dot(aggR, Wl2_ref[HD:, :],
                                  preferred_element_type=jnp.float32)
                        + r2_ref[...])

    return pl.pallas_call(
        body,
        out_shape=jax.ShapeDtypeStruct((N, D), jnp.float32),
    )(S2, C, r2, Wl2)


def kernel(x, edge_index, Wl1, bl1, Wr1, gamma, beta, Wl2, bl2, Wr2):
    src3 = edge_index[0].astype(jnp.int32).reshape(NS, NB, K)
    src3p = src3 + N
    dst3 = edge_index[1].astype(jnp.int32).reshape(NS, NB, K)
    bl1r = bl1.reshape(1, D)
    bl2r = bl2.reshape(1, D)
    gr = gamma.reshape(1, D)
    br = beta.reshape(1, D)

    S1, C = _sc_aggregate(_split_stack(x), src3, src3p, dst3,
                          with_counts=True)
    h2, r2 = _tc_layer1(S1, C, x, Wl1, bl1r, Wr1, gr, br, Wr2, bl2r)
    S2, _ = _sc_aggregate(_split_stack(h2), src3, src3p, dst3,
                          with_counts=False)
    return _tc_layer2(S2, C, r2, Wl2)


# speedup vs baseline: 6.8419x; 1.2592x over previous
import functools

import jax
import jax.numpy as jnp
from jax import lax
from jax.experimental import pallas as pl
from jax.experimental.pallas import tpu as pltpu
from jax.experimental.pallas import tpu_sc as plsc

N = 10000
E = 320000
D = 128
HD = D // 2
NC = 2
NS = 16
EPT = E // NS
K = 80
NB = EPT // K
NPAD = 10240
RPT = NPAD // NS
ZCH = RPT // 5
CW = 16


def _sc_aggregate(x2, src3, src3p, dst3, with_counts):
    mesh = plsc.VectorSubcoreMesh(core_axis_name="c", subcore_axis_name="s")

    @functools.partial(
        pl.kernel,
        out_type=[
            jax.ShapeDtypeStruct((NC, NPAD, HD), jnp.float32),
            jax.ShapeDtypeStruct((NPAD, CW), jnp.float32),
        ],
        mesh=mesh,
        compiler_params=pltpu.CompilerParams(use_tc_tiling_on_sc=False),
        scratch_types=[
            pltpu.VMEM((NB, K), jnp.int32),
            pltpu.VMEM((NB, K), jnp.int32),
            pltpu.VMEM((K, HD), jnp.float32),
            pltpu.VMEM((K, HD), jnp.float32),
            pltpu.VMEM((K, CW), jnp.float32),
            pltpu.VMEM((ZCH, HD), jnp.float32),
            pltpu.VMEM((RPT, CW), jnp.float32),
            pltpu.VMEM_SHARED((NPAD, HD), jnp.float32),
            pltpu.VMEM_SHARED((NPAD, CW), jnp.float32),
            pltpu.SemaphoreType.DMA,
            pltpu.SemaphoreType.DMA,
            pltpu.SemaphoreType.DMA,
            pltpu.SemaphoreType.DMA,
            pltpu.SemaphoreType.DMA,
            pltpu.SemaphoreType.DMA,
        ],
    )
    def agg_kernel(x_hbm, src_hbm, srcp_hbm, dst_hbm, out_hbm, outc_hbm,
                   srcv, dstv, rows0, rows1, ones, zrow, zcnt, acc_s, cnt_s,
                   gsem0, gsem1, ssem0, ssem1, csem0, csem1):
        c = lax.axis_index("c")
        s = lax.axis_index("s")

        def fill_zrow(i, _):
            for j in range(HD // 16):
                zrow[i, pl.ds(j * 16, 16)] = jnp.zeros((16,), jnp.float32)
            return 0
        lax.fori_loop(0, ZCH, fill_zrow, 0)

        if with_counts:
            def fill_zcnt(i, _):
                zcnt[i, :] = jnp.zeros((16,), jnp.float32)
                return 0
            lax.fori_loop(0, RPT, fill_zcnt, 0)

            def fill_ones(i, _):
                ones[i, :] = jnp.ones((16,), jnp.float32)
                return 0
            lax.fori_loop(0, K, fill_ones, 0)

        base = s * RPT
        for z in range(5):
            pltpu.sync_copy(zrow, acc_s.at[pl.ds(base + z * ZCH, ZCH)])
        if with_counts:
            @pl.when(c == 0)
            def _():
                pltpu.sync_copy(zcnt, cnt_s.at[pl.ds(base, RPT)])

        @pl.when(c == 0)
        def _():
            pltpu.sync_copy(src_hbm.at[s], srcv)

        @pl.when(c == 1)
        def _():
            pltpu.sync_copy(srcp_hbm.at[s], srcv)

        pltpu.sync_copy(dst_hbm.at[s], dstv)

        plsc.subcore_barrier()

        def g_start(i, buf, sem):
            pltpu.async_copy(x_hbm.at[srcv.at[i]], buf, sem)

        def g_wait(buf, sem):
            pltpu.make_async_copy(x_hbm.at[srcv.at[0]], buf, sem).wait()

        def s_start(buf, i, sem):
            pltpu.async_copy(buf, acc_s.at[dstv.at[i]], sem, add=True)

        def s_wait(buf, sem):
            pltpu.make_async_copy(buf, acc_s.at[dstv.at[0]], sem).wait()

        def cnt_fire(j, i, sem):
            @pl.when(c == 0)
            def _():
                @pl.when(j > 0)
                def _():
                    pltpu.make_async_copy(ones, cnt_s.at[dstv.at[0]],
                                          sem).wait()
                pltpu.async_copy(ones, cnt_s.at[dstv.at[i]], sem, add=True)

        g_start(0, rows0, gsem0)

        def body(j, _):
            i0 = 2 * j
            i1 = i0 + 1
            g_wait(rows0, gsem0)
            s_start(rows0, i0, ssem0)

            @pl.when(j > 0)
            def _():
                s_wait(rows1, ssem1)
            g_start(i1, rows1, gsem1)
            if with_counts:
                cnt_fire(j, i0, csem0)

            g_wait(rows1, gsem1)
            s_start(rows1, i1, ssem1)
            s_wait(rows0, ssem0)

            @pl.when(j < NB // 2 - 1)
            def _():
                g_start(i0 + 2, rows0, gsem0)
            if with_counts:
                cnt_fire(j, i1, csem1)
            return 0
        lax.fori_loop(0, NB // 2, body, 0)

        s_wait(rows1, ssem1)
        if with_counts:
            @pl.when(c == 0)
            def _():
                pltpu.make_async_copy(ones, cnt_s.at[dstv.at[0]],
                                      csem0).wait()
                pltpu.make_async_copy(ones, cnt_s.at[dstv.at[0]],
                                      csem1).wait()

        plsc.subcore_barrier()
        pltpu.sync_copy(acc_s.at[pl.ds(base, RPT)],
                        out_hbm.at[c, pl.ds(base, RPT)])
        if with_counts:
            @pl.when(c == 0)
            def _():
                pltpu.sync_copy(cnt_s.at[pl.ds(base, RPT)],
                                outc_hbm.at[pl.ds(base, RPT)])

    return agg_kernel(x2, src3, src3p, dst3)


def _split_stack(h):
    return jnp.concatenate([h[:, :HD], h[:, HD:]], axis=0)


def _tc_layer1(S, C, x, Wl1, bl1, Wr1, gamma, beta, Wr2, bl2):
    def body(S_ref, C_ref, x_ref, Wl1_ref, bl1_ref, Wr1_ref, g_ref, b_ref,
             Wr2_ref, bl2_ref, h2_ref, r2_ref):
        inv = 1.0 / jnp.maximum(C_ref[:N, 0:1], 1.0)
        aggL = S_ref[0, :N, :] * inv
        aggR = S_ref[1, :N, :] * inv
        h = (jnp.dot(aggL, Wl1_ref[:HD, :],
                     preferred_element_type=jnp.float32)
             + jnp.dot(aggR, Wl1_ref[HD:, :],
                       preferred_element_type=jnp.float32)
             + jnp.dot(x_ref[...], Wr1_ref[...],
                       preferred_element_type=jnp.float32)
             + bl1_ref[...])
        mu = jnp.mean(h, axis=0, keepdims=True)
        var = jnp.mean((h - mu) * (h - mu), axis=0, keepdims=True)
        hn = (h - mu) / jnp.sqrt(var + 1e-5) * g_ref[...] + b_ref[...]
        h2 = jnp.where(hn >= 0, hn, 0.01 * hn)
        h2_ref[...] = h2
        r2_ref[...] = (jnp.dot(h2, Wr2_ref[...],
                               preferred_element_type=jnp.float32)
                       + bl2_ref[...])

    return pl.pallas_call(
        body,
        out_shape=[
            jax.ShapeDtypeStruct((N, D), jnp.float32),
            jax.ShapeDtypeStruct((N, D), jnp.float32),
        ],
    )(S, C, x, Wl1, bl1, Wr1, gamma, beta, Wr2, bl2)


def _tc_layer2(S2, C, r2, Wl2):
    def body(S_ref, C_ref, r2_ref, Wl2_ref, out_ref):
        inv = 1.0 / jnp.maximum(C_ref[:N, 0:1], 1.0)
        aggL = S_ref[0, :N, :] * inv
        aggR = S_ref[1, :N, :] * inv
        out_ref[...] = (jnp.dot(aggL, Wl2_ref[:HD, :],
                                preferred_element_type=jnp.float32)
                        + jnp.dot(aggR, Wl2_ref[HD:, :],
                                  preferred_element_type=jnp.float32)
                        + r2_ref[...])

    return pl.pallas_call(
        body,
        out_shape=jax.ShapeDtypeStruct((N, D), jnp.float32),
    )(S2, C, r2, Wl2)


def kernel(x, edge_index, Wl1, bl1, Wr1, gamma, beta, Wl2, bl2, Wr2):
    src3 = edge_index[0].astype(jnp.int32).reshape(NS, NB, K)
    src3p = src3 + N
    dst3 = edge_index[1].astype(jnp.int32).reshape(NS, NB, K)
    bl1r = bl1.reshape(1, D)
    bl2r = bl2.reshape(1, D)
    gr = gamma.reshape(1, D)
    br = beta.reshape(1, D)

    S1, C = _sc_aggregate(_split_stack(x), src3, src3p, dst3,
                          with_counts=True)
    h2, r2 = _tc_layer1(S1, C, x, Wl1, bl1r, Wr1, gr, br, Wr2, bl2r)
    S2, _ = _sc_aggregate(_split_stack(h2), src3, src3p, dst3,
                          with_counts=False)
    return _tc_layer2(S2, C, r2, Wl2)
